# Initial kernel scaffold; baseline (speedup 1.0000x reference)
import jax
import jax.numpy as jnp
from jax.experimental import pallas as pl


def kernel(x, edge_index, batch, W11, b11, W12, b12, W21, b21, W22, b22, W31, b31, W32, b32, Wf1, bf1, Wf2, bf2):
    raise NotImplementedError("write your pallas kernel here")


# speedup vs baseline: 4.5360x; 4.5360x over previous
import functools

import jax
import jax.numpy as jnp
from jax import lax
from jax.experimental import pallas as pl
from jax.experimental.pallas import tpu as pltpu
from jax.experimental.pallas import tpu_sc as plsc

N_NODES = 10000
FEAT = 128
NGROUP = 64
NCORES = 2
NSUB = 16
EDGE_CHUNK = 80
ROW_BLK = 1000
NBLK = N_NODES // ROW_BLK


def _agg_call(x, src, dst, zeros):
    E = src.shape[0]
    per_worker = E // (NCORES * NSUB)
    iters = per_worker // EDGE_CHUNK
    rows_per_tile = (N_NODES // NSUB) // 8 * 8
    tail_rows = N_NODES - rows_per_tile * NSUB
    mesh = plsc.VectorSubcoreMesh(core_axis_name="c", subcore_axis_name="s")

    @functools.partial(
        pl.kernel,
        mesh=mesh,
        out_type=jax.ShapeDtypeStruct((NCORES, N_NODES, FEAT), jnp.float32),
        scratch_types=[
            pltpu.VMEM((EDGE_CHUNK,), jnp.int32),
            pltpu.VMEM((EDGE_CHUNK,), jnp.int32),
            pltpu.VMEM((EDGE_CHUNK, FEAT), jnp.float32),
            pltpu.VMEM_SHARED((N_NODES, FEAT), jnp.float32),
        ],
    )
    def agg(x_hbm, src_hbm, dst_hbm, z_hbm, out_hbm, src_v, dst_v, rows_v, acc_sh):
        c = lax.axis_index("c")
        s = lax.axis_index("s")
        r0 = s * rows_per_tile
        pltpu.sync_copy(z_hbm.at[pl.ds(r0, rows_per_tile)],
                        acc_sh.at[pl.ds(r0, rows_per_tile)])

        @pl.when(s == NSUB - 1)
        def _():
            t0 = NSUB * rows_per_tile
            pltpu.sync_copy(z_hbm.at[pl.ds(t0, tail_rows)],
                            acc_sh.at[pl.ds(t0, tail_rows)])

        plsc.subcore_barrier()
        e0 = (c * NSUB + s) * per_worker

        def body(i, carry):
            base = e0 + i * EDGE_CHUNK
            pltpu.sync_copy(src_hbm.at[pl.ds(base, EDGE_CHUNK)], src_v)
            pltpu.sync_copy(dst_hbm.at[pl.ds(base, EDGE_CHUNK)], dst_v)
            pltpu.sync_copy(x_hbm.at[src_v], rows_v)
            pltpu.sync_copy(rows_v, acc_sh.at[dst_v], add=True)
            return carry

        lax.fori_loop(0, iters, body, 0)
        plsc.subcore_barrier()
        pltpu.sync_copy(acc_sh.at[pl.ds(r0, rows_per_tile)],
                        out_hbm.at[c, pl.ds(r0, rows_per_tile)])

        @pl.when(s == NSUB - 1)
        def _():
            t0 = NSUB * rows_per_tile
            pltpu.sync_copy(acc_sh.at[pl.ds(t0, tail_rows)],
                            out_hbm.at[c, pl.ds(t0, tail_rows)])

    return agg(x, src, dst, zeros)


def _conv_mlp_call(x, p, W1, b1, W2, b2):
    def body(x_ref, p_ref, w1_ref, b1_ref, w2_ref, b2_ref, o_ref):
        h = x_ref[...] + p_ref[0] + p_ref[1]
        t = jnp.maximum(
            jnp.dot(h, w1_ref[...], preferred_element_type=jnp.float32)
            + b1_ref[...], 0.0)
        o = (jnp.dot(t, w2_ref[...], preferred_element_type=jnp.float32)
             + b2_ref[...])
        o_ref[...] = jnp.maximum(o, 0.0)

    return pl.pallas_call(
        body,
        grid=(NBLK,),
        in_specs=[
            pl.BlockSpec((ROW_BLK, FEAT), lambda i: (i, 0)),
            pl.BlockSpec((NCORES, ROW_BLK, FEAT), lambda i: (0, i, 0)),
            pl.BlockSpec((FEAT, FEAT), lambda i: (0, 0)),
            pl.BlockSpec((1, FEAT), lambda i: (0, 0)),
            pl.BlockSpec((FEAT, FEAT), lambda i: (0, 0)),
            pl.BlockSpec((1, FEAT), lambda i: (0, 0)),
        ],
        out_specs=pl.BlockSpec((ROW_BLK, FEAT), lambda i: (i, 0)),
        out_shape=jax.ShapeDtypeStruct((N_NODES, FEAT), jnp.float32),
    )(x, p, W1, b1.reshape(1, FEAT), W2, b2.reshape(1, FEAT))


def _conv3_pool_head_call(x, p, W1, b1, W2, b2, batch_r,
                          Wf1, bf1, Wf2p, bf2p):
    def body(x_ref, p_ref, w1_ref, b1_ref, w2_ref, b2_ref, batch_ref,
             hw1_ref, hb1_ref, hw2_ref, hb2_ref, head_ref, acc_ref):
        i = pl.program_id(0)

        @pl.when(i == 0)
        def _():
            acc_ref[...] = jnp.full((NGROUP, FEAT), -jnp.inf, jnp.float32)

        h = x_ref[...] + p_ref[0] + p_ref[1]
        t = jnp.maximum(
            jnp.dot(h, w1_ref[...], preferred_element_type=jnp.float32)
            + b1_ref[...], 0.0)
        o = (jnp.dot(t, w2_ref[...], preferred_element_type=jnp.float32)
             + b2_ref[...])

        b = batch_ref[...]
        glo = jnp.min(b)
        ghi = jnp.max(b)

        def gbody(g, acc):
            m = b == g
            mx = jnp.max(jnp.where(m, o, -jnp.inf), axis=0, keepdims=True)
            rowm = lax.broadcasted_iota(jnp.int32, (NGROUP, 1), 0) == g
            return jnp.where(rowm, jnp.maximum(acc, mx), acc)

        acc_ref[...] = lax.fori_loop(glo, ghi + 1, gbody, acc_ref[...])

        @pl.when(i == NBLK - 1)
        def _():
            th = jnp.maximum(
                jnp.dot(acc_ref[...], hw1_ref[...],
                        preferred_element_type=jnp.float32) + hb1_ref[...],
                0.0)
            head_ref[...] = (
                jnp.dot(th, hw2_ref[...], preferred_element_type=jnp.float32)
                + hb2_ref[...])

    return pl.pallas_call(
        body,
        grid=(NBLK,),
        in_specs=[
            pl.BlockSpec((ROW_BLK, FEAT), lambda i: (i, 0)),
            pl.BlockSpec((NCORES, ROW_BLK, FEAT), lambda i: (0, i, 0)),
            pl.BlockSpec((FEAT, FEAT), lambda i: (0, 0)),
            pl.BlockSpec((1, FEAT), lambda i: (0, 0)),
            pl.BlockSpec((FEAT, FEAT), lambda i: (0, 0)),
            pl.BlockSpec((1, FEAT), lambda i: (0, 0)),
            pl.BlockSpec((ROW_BLK, FEAT), lambda i: (i, 0)),
            pl.BlockSpec((FEAT, FEAT), lambda i: (0, 0)),
            pl.BlockSpec((1, FEAT), lambda i: (0, 0)),
            pl.BlockSpec((FEAT, FEAT), lambda i: (0, 0)),
            pl.BlockSpec((1, FEAT), lambda i: (0, 0)),
        ],
        out_specs=pl.BlockSpec((NGROUP, FEAT), lambda i: (0, 0)),
        out_shape=jax.ShapeDtypeStruct((NGROUP, FEAT), jnp.float32),
        scratch_shapes=[pltpu.VMEM((NGROUP, FEAT), jnp.float32)],
    )(x, p, W1, b1.reshape(1, FEAT), W2, b2.reshape(1, FEAT), batch_r,
      Wf1, bf1.reshape(1, FEAT), Wf2p, bf2p)


def kernel(x, edge_index, batch, W11, b11, W12, b12, W21, b21, W22, b22,
           W31, b31, W32, b32, Wf1, bf1, Wf2, bf2):
    src = edge_index[0]
    dst = edge_index[1]
    zeros = jnp.zeros((N_NODES, FEAT), jnp.float32)

    p1 = _agg_call(x, src, dst, zeros)
    h1 = _conv_mlp_call(x, p1, W11, b11, W12, b12)
    p2 = _agg_call(h1, src, dst, zeros)
    h2 = _conv_mlp_call(h1, p2, W21, b21, W22, b22)
    p3 = _agg_call(h2, src, dst, zeros)

    batch_r = jnp.broadcast_to(batch[:, None], (N_NODES, FEAT))
    C = Wf2.shape[1]
    Wf2p = jnp.pad(Wf2, ((0, 0), (0, FEAT - C)))
    bf2p = jnp.pad(bf2, (0, FEAT - C)).reshape(1, FEAT)
    head = _conv3_pool_head_call(h2, p3, W31, b31, W32, b32, batch_r,
                                 Wf1, bf1, Wf2p, bf2p)
    return head[:, :C]


# speedup vs baseline: 8.2470x; 1.8181x over previous
import functools

import jax
import jax.numpy as jnp
from jax import lax
from jax.experimental import pallas as pl
from jax.experimental.pallas import tpu as pltpu
from jax.experimental.pallas import tpu_sc as plsc

N_NODES = 10000
FEAT = 128
NGROUP = 64
NCORES = 2
NSUB = 16
EDGE_CHUNK = 80
ROW_BLK = 1000
NBLK = N_NODES // ROW_BLK


def _agg_call(x, src1, dst3, zeros):
    CHUNKS = dst3.shape[1]
    rows_per_tile = (N_NODES // NSUB) // 8 * 8
    tail_rows = N_NODES - rows_per_tile * NSUB
    mesh = plsc.VectorSubcoreMesh(core_axis_name="c", subcore_axis_name="s")

    @functools.partial(
        pl.kernel,
        mesh=mesh,
        out_type=jax.ShapeDtypeStruct((NCORES, N_NODES, FEAT), jnp.float32),
        scratch_types=[
            pltpu.VMEM((CHUNKS * EDGE_CHUNK,), jnp.int32),
            pltpu.VMEM((CHUNKS, EDGE_CHUNK), jnp.int32),
            pltpu.VMEM((EDGE_CHUNK, FEAT), jnp.float32),
            pltpu.VMEM((EDGE_CHUNK, FEAT), jnp.float32),
            pltpu.SemaphoreType.DMA,
            pltpu.SemaphoreType.DMA,
            pltpu.VMEM_SHARED((N_NODES, FEAT), jnp.float32),
        ],
    )
    def agg(x_hbm, src_hbm, dst_hbm, z_hbm, out_hbm, src_all, dst_all,
            rows0, rows1, gsem0, gsem1, acc_sh):
        c = lax.axis_index("c")
        s = lax.axis_index("s")
        r0 = s * rows_per_tile
        pltpu.sync_copy(z_hbm.at[pl.ds(r0, rows_per_tile)],
                        acc_sh.at[pl.ds(r0, rows_per_tile)])

        @pl.when(s == NSUB - 1)
        def _():
            t0 = NSUB * rows_per_tile
            pltpu.sync_copy(z_hbm.at[pl.ds(t0, tail_rows)],
                            acc_sh.at[pl.ds(t0, tail_rows)])

        w = c * NSUB + s
        pltpu.sync_copy(src_hbm.at[pl.ds(w * CHUNKS * EDGE_CHUNK,
                                         CHUNKS * EDGE_CHUNK)], src_all)
        pltpu.sync_copy(dst_hbm.at[w], dst_all)
        plsc.subcore_barrier()

        def sidx(j):
            return src_all.at[pl.ds(j * EDGE_CHUNK, EDGE_CHUNK)]

        pltpu.async_copy(x_hbm.at[sidx(0)], rows0, gsem0)

        def body(g, carry):
            j = 2 * g
            pltpu.make_async_copy(x_hbm.at[sidx(j)], rows0, gsem0).wait()
            pltpu.async_copy(x_hbm.at[sidx(j + 1)], rows1, gsem1)
            pltpu.sync_copy(rows0, acc_sh.at[dst_all.at[j]], add=True)
            pltpu.make_async_copy(x_hbm.at[sidx(j + 1)], rows1, gsem1).wait()
            pltpu.async_copy(x_hbm.at[sidx(j + 2)], rows0, gsem0)
            pltpu.sync_copy(rows1, acc_sh.at[dst_all.at[j + 1]], add=True)
            return carry

        lax.fori_loop(0, (CHUNKS - 1) // 2, body, 0)
        jt = CHUNKS - 1
        pltpu.make_async_copy(x_hbm.at[sidx(jt)], rows0, gsem0).wait()
        pltpu.sync_copy(rows0, acc_sh.at[dst_all.at[jt]], add=True)

        plsc.subcore_barrier()
        pltpu.sync_copy(acc_sh.at[pl.ds(r0, rows_per_tile)],
                        out_hbm.at[c, pl.ds(r0, rows_per_tile)])

        @pl.when(s == NSUB - 1)
        def _():
            t0 = NSUB * rows_per_tile
            pltpu.sync_copy(acc_sh.at[pl.ds(t0, tail_rows)],
                            out_hbm.at[c, pl.ds(t0, tail_rows)])

    return agg(x, src1, dst3, zeros)


def _conv_mlp_call(x, p, W1, b1, W2, b2):
    def body(x_ref, p_ref, w1_ref, b1_ref, w2_ref, b2_ref, o_ref):
        h = x_ref[...] + p_ref[0] + p_ref[1]
        t = jnp.maximum(
            jnp.dot(h, w1_ref[...], preferred_element_type=jnp.float32)
            + b1_ref[...], 0.0)
        o = (jnp.dot(t, w2_ref[...], preferred_element_type=jnp.float32)
             + b2_ref[...])
        o_ref[...] = jnp.maximum(o, 0.0)

    return pl.pallas_call(
        body,
        grid=(NBLK,),
        in_specs=[
            pl.BlockSpec((ROW_BLK, FEAT), lambda i: (i, 0)),
            pl.BlockSpec((NCORES, ROW_BLK, FEAT), lambda i: (0, i, 0)),
            pl.BlockSpec((FEAT, FEAT), lambda i: (0, 0)),
            pl.BlockSpec((1, FEAT), lambda i: (0, 0)),
            pl.BlockSpec((FEAT, FEAT), lambda i: (0, 0)),
            pl.BlockSpec((1, FEAT), lambda i: (0, 0)),
        ],
        out_specs=pl.BlockSpec((ROW_BLK, FEAT), lambda i: (i, 0)),
        out_shape=jax.ShapeDtypeStruct((N_NODES, FEAT), jnp.float32),
    )(x, p, W1, b1.reshape(1, FEAT), W2, b2.reshape(1, FEAT))


def _conv3_pool_head_call(x, p, W1, b1, W2, b2, batch_r,
                          Wf1, bf1, Wf2p, bf2p):
    def body(x_ref, p_ref, w1_ref, b1_ref, w2_ref, b2_ref, batch_ref,
             hw1_ref, hb1_ref, hw2_ref, hb2_ref, head_ref, acc_ref):
        i = pl.program_id(0)

        @pl.when(i == 0)
        def _():
            acc_ref[...] = jnp.full((NGROUP, FEAT), -jnp.inf, jnp.float32)

        h = x_ref[...] + p_ref[0] + p_ref[1]
        t = jnp.maximum(
            jnp.dot(h, w1_ref[...], preferred_element_type=jnp.float32)
            + b1_ref[...], 0.0)
        o = (jnp.dot(t, w2_ref[...], preferred_element_type=jnp.float32)
             + b2_ref[...])

        b = batch_ref[...]
        glo = jnp.min(b)
        ghi = jnp.max(b)

        def gbody(g, acc):
            m = b == g
            mx = jnp.max(jnp.where(m, o, -jnp.inf), axis=0, keepdims=True)
            rowm = lax.broadcasted_iota(jnp.int32, (NGROUP, 1), 0) == g
            return jnp.where(rowm, jnp.maximum(acc, mx), acc)

        acc_ref[...] = lax.fori_loop(glo, ghi + 1, gbody, acc_ref[...])

        @pl.when(i == NBLK - 1)
        def _():
            th = jnp.maximum(
                jnp.dot(acc_ref[...], hw1_ref[...],
                        preferred_element_type=jnp.float32) + hb1_ref[...],
                0.0)
            head_ref[...] = (
                jnp.dot(th, hw2_ref[...], preferred_element_type=jnp.float32)
                + hb2_ref[...])

    return pl.pallas_call(
        body,
        grid=(NBLK,),
        in_specs=[
            pl.BlockSpec((ROW_BLK, FEAT), lambda i: (i, 0)),
            pl.BlockSpec((NCORES, ROW_BLK, FEAT), lambda i: (0, i, 0)),
            pl.BlockSpec((FEAT, FEAT), lambda i: (0, 0)),
            pl.BlockSpec((1, FEAT), lambda i: (0, 0)),
            pl.BlockSpec((FEAT, FEAT), lambda i: (0, 0)),
            pl.BlockSpec((1, FEAT), lambda i: (0, 0)),
            pl.BlockSpec((ROW_BLK, FEAT), lambda i: (i, 0)),
            pl.BlockSpec((FEAT, FEAT), lambda i: (0, 0)),
            pl.BlockSpec((1, FEAT), lambda i: (0, 0)),
            pl.BlockSpec((FEAT, FEAT), lambda i: (0, 0)),
            pl.BlockSpec((1, FEAT), lambda i: (0, 0)),
        ],
        out_specs=pl.BlockSpec((NGROUP, FEAT), lambda i: (0, 0)),
        out_shape=jax.ShapeDtypeStruct((NGROUP, FEAT), jnp.float32),
        scratch_shapes=[pltpu.VMEM((NGROUP, FEAT), jnp.float32)],
    )(x, p, W1, b1.reshape(1, FEAT), W2, b2.reshape(1, FEAT), batch_r,
      Wf1, bf1.reshape(1, FEAT), Wf2p, bf2p)


def kernel(x, edge_index, batch, W11, b11, W12, b12, W21, b21, W22, b22,
           W31, b31, W32, b32, Wf1, bf1, Wf2, bf2):
    src1 = edge_index[0]
    dst3 = edge_index[1].reshape(NCORES * NSUB, -1, EDGE_CHUNK)
    zeros = jnp.zeros((N_NODES, FEAT), jnp.float32)

    p1 = _agg_call(x, src1, dst3, zeros)
    h1 = _conv_mlp_call(x, p1, W11, b11, W12, b12)
    p2 = _agg_call(h1, src1, dst3, zeros)
    h2 = _conv_mlp_call(h1, p2, W21, b21, W22, b22)
    p3 = _agg_call(h2, src1, dst3, zeros)

    batch_r = jnp.broadcast_to(batch[:, None], (N_NODES, FEAT))
    C = Wf2.shape[1]
    Wf2p = jnp.pad(Wf2, ((0, 0), (0, FEAT - C)))
    bf2p = jnp.pad(bf2, (0, FEAT - C)).reshape(1, FEAT)
    head = _conv3_pool_head_call(h2, p3, W31, b31, W32, b32, batch_r,
                                 Wf1, bf1, Wf2p, bf2p)
    return head[:, :C]


# speedup vs baseline: 11.5900x; 1.4054x over previous
import functools

import jax
import jax.numpy as jnp
from jax import lax
from jax.experimental import pallas as pl
from jax.experimental.pallas import tpu as pltpu
from jax.experimental.pallas import tpu_sc as plsc

N_NODES = 10000
FEAT = 128
NGROUP = 64
NCORES = 2
NSUB = 16
GCHUNK = 40
DSTROW = 80
CPR = DSTROW // GCHUNK
DEPTH = 4
ROW_BLK = 1000
NBLK = N_NODES // ROW_BLK


def _agg_call(x, src1, dst3, zeros):
    CHUNKS = dst3.shape[1] * CPR
    rows_per_tile = (N_NODES // NSUB) // 8 * 8
    tail_rows = N_NODES - rows_per_tile * NSUB
    mesh = plsc.VectorSubcoreMesh(core_axis_name="c", subcore_axis_name="s")

    @functools.partial(
        pl.kernel,
        mesh=mesh,
        out_type=jax.ShapeDtypeStruct((NCORES, N_NODES, FEAT), jnp.float32),
        scratch_types=[
            pltpu.VMEM((CHUNKS * GCHUNK,), jnp.int32),
            pltpu.VMEM((CHUNKS // CPR, DSTROW), jnp.int32),
        ] + [pltpu.VMEM((GCHUNK, FEAT), jnp.float32)] * DEPTH
          + [pltpu.SemaphoreType.DMA] * DEPTH
          + [pltpu.VMEM_SHARED((N_NODES, FEAT), jnp.float32)],
    )
    def agg(x_hbm, src_hbm, dst_hbm, z_hbm, out_hbm, src_all, dst_all,
            *ring_and_acc):
        bufs = ring_and_acc[:DEPTH]
        sems = ring_and_acc[DEPTH:2 * DEPTH]
        acc_sh = ring_and_acc[2 * DEPTH]
        c = lax.axis_index("c")
        s = lax.axis_index("s")
        r0 = s * rows_per_tile
        pltpu.sync_copy(z_hbm.at[pl.ds(r0, rows_per_tile)],
                        acc_sh.at[pl.ds(r0, rows_per_tile)])

        @pl.when(s == NSUB - 1)
        def _():
            t0 = NSUB * rows_per_tile
            pltpu.sync_copy(z_hbm.at[pl.ds(t0, tail_rows)],
                            acc_sh.at[pl.ds(t0, tail_rows)])

        w = c * NSUB + s
        pltpu.sync_copy(src_hbm.at[pl.ds(w * CHUNKS * GCHUNK,
                                         CHUNKS * GCHUNK)], src_all)
        pltpu.sync_copy(dst_hbm.at[w], dst_all)
        plsc.subcore_barrier()

        def sidx(j):
            return src_all.at[pl.ds(j * GCHUNK, GCHUNK)]

        def didx(row, half):
            return dst_all.at[row, pl.ds(half * GCHUNK, GCHUNK)]

        def issue(j, d):
            pltpu.async_copy(x_hbm.at[sidx(j)], bufs[d], sems[d])

        def wait(j, d):
            pltpu.make_async_copy(x_hbm.at[sidx(j)], bufs[d], sems[d]).wait()

        def scat(d, row, half):
            pltpu.sync_copy(bufs[d], acc_sh.at[didx(row, half)], add=True)

        for q in range(DEPTH - 1):
            issue(q, q)

        def body(g, carry):
            j = DEPTH * g
            for d in range(DEPTH):
                wait(j + d, d)
                issue(j + d + DEPTH - 1, (d + DEPTH - 1) % DEPTH)
                scat(d, 2 * g + d // CPR, d % CPR)
            return carry

        T = (CHUNKS - DEPTH + 1) // DEPTH
        lax.fori_loop(0, T, body, 0)
        for q in range(DEPTH * T, CHUNKS):
            wait(q, q % DEPTH)
            if q + DEPTH - 1 < CHUNKS:
                issue(q + DEPTH - 1, (q + DEPTH - 1) % DEPTH)
            scat(q % DEPTH, q // CPR, q % CPR)

        plsc.subcore_barrier()
        pltpu.sync_copy(acc_sh.at[pl.ds(r0, rows_per_tile)],
                        out_hbm.at[c, pl.ds(r0, rows_per_tile)])

        @pl.when(s == NSUB - 1)
        def _():
            t0 = NSUB * rows_per_tile
            pltpu.sync_copy(acc_sh.at[pl.ds(t0, tail_rows)],
                            out_hbm.at[c, pl.ds(t0, tail_rows)])

    return agg(x, src1, dst3, zeros)


def _conv_mlp_call(x, p, W1, b1, W2, b2):
    def body(x_ref, p_ref, w1_ref, b1_ref, w2_ref, b2_ref, o_ref):
        h = x_ref[...] + p_ref[0] + p_ref[1]
        t = jnp.maximum(
            jnp.dot(h, w1_ref[...], preferred_element_type=jnp.float32)
            + b1_ref[...], 0.0)
        o = (jnp.dot(t, w2_ref[...], preferred_element_type=jnp.float32)
             + b2_ref[...])
        o_ref[...] = jnp.maximum(o, 0.0)

    return pl.pallas_call(
        body,
        grid=(NBLK,),
        in_specs=[
            pl.BlockSpec((ROW_BLK, FEAT), lambda i: (i, 0)),
            pl.BlockSpec((NCORES, ROW_BLK, FEAT), lambda i: (0, i, 0)),
            pl.BlockSpec((FEAT, FEAT), lambda i: (0, 0)),
            pl.BlockSpec((1, FEAT), lambda i: (0, 0)),
            pl.BlockSpec((FEAT, FEAT), lambda i: (0, 0)),
            pl.BlockSpec((1, FEAT), lambda i: (0, 0)),
        ],
        out_specs=pl.BlockSpec((ROW_BLK, FEAT), lambda i: (i, 0)),
        out_shape=jax.ShapeDtypeStruct((N_NODES, FEAT), jnp.float32),
    )(x, p, W1, b1.reshape(1, FEAT), W2, b2.reshape(1, FEAT))


def _conv3_pool_head_call(x, p, W1, b1, W2, b2, batch_r,
                          Wf1, bf1, Wf2p, bf2p):
    def body(x_ref, p_ref, w1_ref, b1_ref, w2_ref, b2_ref, batch_ref,
             hw1_ref, hb1_ref, hw2_ref, hb2_ref, head_ref, acc_ref):
        i = pl.program_id(0)

        @pl.when(i == 0)
        def _():
            acc_ref[...] = jnp.full((NGROUP, FEAT), -jnp.inf, jnp.float32)

        h = x_ref[...] + p_ref[0] + p_ref[1]
        t = jnp.maximum(
            jnp.dot(h, w1_ref[...], preferred_element_type=jnp.float32)
            + b1_ref[...], 0.0)
        o = (jnp.dot(t, w2_ref[...], preferred_element_type=jnp.float32)
             + b2_ref[...])

        b = batch_ref[...]
        glo = jnp.min(b)
        ghi = jnp.max(b)

        def gbody(g, acc):
            m = b == g
            mx = jnp.max(jnp.where(m, o, -jnp.inf), axis=0, keepdims=True)
            rowm = lax.broadcasted_iota(jnp.int32, (NGROUP, 1), 0) == g
            return jnp.where(rowm, jnp.maximum(acc, mx), acc)

        acc_ref[...] = lax.fori_loop(glo, ghi + 1, gbody, acc_ref[...])

        @pl.when(i == NBLK - 1)
        def _():
            th = jnp.maximum(
                jnp.dot(acc_ref[...], hw1_ref[...],
                        preferred_element_type=jnp.float32) + hb1_ref[...],
                0.0)
            head_ref[...] = (
                jnp.dot(th, hw2_ref[...], preferred_element_type=jnp.float32)
                + hb2_ref[...])

    return pl.pallas_call(
        body,
        grid=(NBLK,),
        in_specs=[
            pl.BlockSpec((ROW_BLK, FEAT), lambda i: (i, 0)),
            pl.BlockSpec((NCORES, ROW_BLK, FEAT), lambda i: (0, i, 0)),
            pl.BlockSpec((FEAT, FEAT), lambda i: (0, 0)),
            pl.BlockSpec((1, FEAT), lambda i: (0, 0)),
            pl.BlockSpec((FEAT, FEAT), lambda i: (0, 0)),
            pl.BlockSpec((1, FEAT), lambda i: (0, 0)),
            pl.BlockSpec((ROW_BLK, FEAT), lambda i: (i, 0)),
            pl.BlockSpec((FEAT, FEAT), lambda i: (0, 0)),
            pl.BlockSpec((1, FEAT), lambda i: (0, 0)),
            pl.BlockSpec((FEAT, FEAT), lambda i: (0, 0)),
            pl.BlockSpec((1, FEAT), lambda i: (0, 0)),
        ],
        out_specs=pl.BlockSpec((NGROUP, FEAT), lambda i: (0, 0)),
        out_shape=jax.ShapeDtypeStruct((NGROUP, FEAT), jnp.float32),
        scratch_shapes=[pltpu.VMEM((NGROUP, FEAT), jnp.float32)],
    )(x, p, W1, b1.reshape(1, FEAT), W2, b2.reshape(1, FEAT), batch_r,
      Wf1, bf1.reshape(1, FEAT), Wf2p, bf2p)


def kernel(x, edge_index, batch, W11, b11, W12, b12, W21, b21, W22, b22,
           W31, b31, W32, b32, Wf1, bf1, Wf2, bf2):
    src1 = edge_index[0]
    dst3 = edge_index[1].reshape(NCORES * NSUB, -1, DSTROW)
    zeros = jnp.zeros((N_NODES, FEAT), jnp.float32)

    p1 = _agg_call(x, src1, dst3, zeros)
    h1 = _conv_mlp_call(x, p1, W11, b11, W12, b12)
    p2 = _agg_call(h1, src1, dst3, zeros)
    h2 = _conv_mlp_call(h1, p2, W21, b21, W22, b22)
    p3 = _agg_call(h2, src1, dst3, zeros)

    batch_r = jnp.broadcast_to(batch[:, None], (N_NODES, FEAT))
    C = Wf2.shape[1]
    Wf2p = jnp.pad(Wf2, ((0, 0), (0, FEAT - C)))
    bf2p = jnp.pad(bf2, (0, FEAT - C)).reshape(1, FEAT)
    head = _conv3_pool_head_call(h2, p3, W31, b31, W32, b32, batch_r,
                                 Wf1, bf1, Wf2p, bf2p)
    return head[:, :C]
